# baseline (speedup 1.0000x reference)
import functools

import jax
import jax.numpy as jnp
from jax import lax
from jax.experimental import pallas as pl
from jax.experimental.pallas import tpu as pltpu
from jax.experimental.pallas import tpu_sc as plsc

D = 64
CHK = 256


def _make_gather(B: int, H: int, V: int):
    info = plsc.get_sparse_core_info()
    NC, NS, L = info.num_cores, info.num_subcores, info.num_lanes
    NW = NC * NS
    assert B % NW == 0
    rows_per_w = B // NW
    half = rows_per_w // 2
    n_full = V // CHK
    per_tile = (n_full // NS) * NS
    nk = per_tile // NS
    mesh = plsc.VectorSubcoreMesh(core_axis_name="c", subcore_axis_name="s")

    @functools.partial(
        pl.kernel,
        mesh=mesh,
        out_type=jax.ShapeDtypeStruct((B, H, D), jnp.float32),
        scratch_types=[
            pltpu.HBM((NC, V, D), jnp.float32),
            pltpu.VMEM((2, CHK, D), jnp.float32),
            pltpu.VMEM((rows_per_w // 4, H), jnp.int32),
            pltpu.VMEM((2, H, D), jnp.float32),
            pltpu.SemaphoreType.DMA,
            pltpu.SemaphoreType.DMA,
            pltpu.SemaphoreType.DMA,
            pltpu.SemaphoreType.DMA,
            pltpu.SemaphoreType.DMA,
            pltpu.SemaphoreType.DMA,
            pltpu.SemaphoreType.DMA,
        ],
        compiler_params=pltpu.CompilerParams(use_tc_tiling_on_sc=True),
    )
    def gather_kernel(x_hbm, tab_hbm, out_hbm, scr, vt, idx2d, rows_v,
                      rsem0, rsem1, wsem0, wsem1, gsem, ssem0, ssem1):
        cid = lax.axis_index("c")
        sid = lax.axis_index("s")
        wid = sid * NC + cid
        base = wid * rows_per_w
        myscr = scr.at[cid]

        rsems = (rsem0, rsem1)
        wsems = (wsem0, wsem1)

        def g_of(k):
            return k * NS + sid

        def fire_read(k, p):
            pltpu.async_copy(tab_hbm.at[pl.ds(g_of(k) * CHK, CHK)],
                             vt.at[p], rsems[p])

        def drain_read(p):
            pltpu.make_async_copy(tab_hbm.at[pl.ds(0, CHK)], vt.at[p],
                                  rsems[p]).wait()

        def fire_write(k, p):
            pltpu.async_copy(vt.at[p], myscr.at[pl.ds(g_of(k) * CHK, CHK)],
                             wsems[p])

        def drain_write(p):
            pltpu.make_async_copy(vt.at[p], myscr.at[pl.ds(0, CHK)],
                                  wsems[p]).wait()

        def step(k, p, fire_next, wait_write):
            if fire_next:
                if wait_write:
                    drain_write(1 - p)
                fire_read(k + 1, 1 - p)
            drain_read(p)
            fire_write(k, p)

        fire_read(0, 0)
        step(0, 0, True, False)
        step(1, 1, True, True)

        def ph1_body(p2, carry):
            step(2 * p2, 0, True, True)
            step(2 * p2 + 1, 1, True, True)
            return carry

        lax.fori_loop(1, nk // 2 - 1, ph1_body, 0, unroll=False)
        step(nk - 2, 0, True, True)
        step(nk - 1, 1, False, False)
        drain_write(0)
        drain_write(1)

        @pl.when(sid == 0)
        def _():
            rest = V - per_tile * CHK
            pieces = []
            off0 = 0
            while off0 < rest:
                pieces.append((off0, min(CHK, rest - off0)))
                off0 += CHK
            for off, n in pieces:
                src = tab_hbm.at[pl.ds(per_tile * CHK + off, n)]
                pltpu.sync_copy(src, vt.at[0].at[pl.ds(0, n)])
                pltpu.sync_copy(vt.at[0].at[pl.ds(0, n)],
                                myscr.at[pl.ds(per_tile * CHK + off, n)])

        plsc.subcore_barrier()

        quarter = rows_per_w // 4

        def stage_idx(q):
            pltpu.sync_copy(x_hbm.at[pl.ds(base + q * quarter, quarter)],
                            idx2d)

        stage_idx(0)
        ssems = (ssem0, ssem1)

        def run_row(b_loc, b_glob, buf, wait_store):
            store_src = rows_v.at[buf]
            store_dst = out_hbm.at[base + b_glob]
            if wait_store:
                pltpu.make_async_copy(store_src, store_dst, ssems[buf]).wait()
            d1 = pltpu.async_copy(
                myscr.at[idx2d.at[b_loc, pl.ds(0, 128)]],
                rows_v.at[buf].at[pl.ds(0, 128)], gsem)
            d2 = pltpu.async_copy(
                myscr.at[idx2d.at[b_loc, pl.ds(128, H - 128)]],
                rows_v.at[buf].at[pl.ds(128, H - 128)], gsem)
            d1.wait()
            d2.wait()
            pltpu.async_copy(store_src, store_dst, ssems[buf])

        run_row(0, 0, 0, False)
        run_row(1, 1, 1, False)

        def body_a(p, carry):
            run_row(2 * p, 2 * p, 0, True)
            run_row(2 * p + 1, 2 * p + 1, 1, True)
            return carry

        lax.fori_loop(1, quarter // 2, body_a, 0, unroll=False)

        for q in range(1, 4):
            stage_idx(q)

            def body_q(p, carry, _q=q):
                run_row(2 * p, _q * quarter + 2 * p, 0, True)
                run_row(2 * p + 1, _q * quarter + 2 * p + 1, 1, True)
                return carry

            lax.fori_loop(0, quarter // 2, body_q, 0, unroll=False)

        pltpu.make_async_copy(
            rows_v.at[0], out_hbm.at[base + rows_per_w - 2], ssems[0]).wait()
        pltpu.make_async_copy(
            rows_v.at[1], out_hbm.at[base + rows_per_w - 1], ssems[1]).wait()

    return gather_kernel


def kernel(x, table):
    B, H = x.shape
    V = table.shape[0]
    return _make_gather(B, H, V)(x.astype(jnp.int32), table)
